# baseline (speedup 1.0000x reference)
import functools

import jax
import jax.numpy as jnp
from jax import lax
from jax.experimental import pallas as pl
from jax.experimental.pallas import tpu as pltpu
from jax.experimental.pallas import tpu_sc as plsc

NC = 2
NS = 16
NW = NC * NS
CHUNK = 128
L = 16


@functools.partial(jax.jit, static_argnums=(2,))
def _gather(ids, table, batch):
    row = 32
    gper = 128 // row
    b_per_w = batch // NW
    n_chunks = b_per_w // CHUNK
    out_rows = batch // gper
    or_per_w = out_rows // NW
    mesh = plsc.VectorSubcoreMesh(
        core_axis_name="c", subcore_axis_name="s",
        num_cores=NC, num_subcores=NS)

    @functools.partial(
        pl.kernel,
        out_type=jax.ShapeDtypeStruct((out_rows, 128), jnp.float32),
        mesh=mesh,
        scratch_types=[
            pltpu.VMEM((b_per_w,), jnp.int32),
            pltpu.VMEM((n_chunks, CHUNK), jnp.int32),
            pltpu.VMEM((b_per_w,), jnp.int32),
            pltpu.VMEM((b_per_w, 128), jnp.float32),
            pltpu.VMEM((or_per_w, 128), jnp.float32),
            pltpu.SemaphoreType.DMA,
        ],
        compiler_params=pltpu.CompilerParams(needs_layout_passes=False),
    )
    def k(ids_hbm, table_hbm, out_hbm, ids_v, grp2_v, sub_v, g_v, o_v, sem):
        wid = lax.axis_index("s") * NC + lax.axis_index("c")
        pltpu.sync_copy(ids_hbm.at[pl.ds(wid * b_per_w, b_per_w)], ids_v)

        iota = lax.iota(jnp.int32, L)
        for t in range(b_per_w // L):
            v = ids_v[pl.ds(t * L, L)]
            grp2_v[t // (CHUNK // L), pl.ds((t % (CHUNK // L)) * L, L)] = lax.shift_right_logical(v, 2)
            sub_v[pl.ds(t * L, L)] = lax.bitwise_and(v, 3)

        copies = []
        for j in range(n_chunks):
            copies.append(pltpu.async_copy(
                table_hbm.at[grp2_v.at[j]],
                g_v.at[pl.ds(j * CHUNK, CHUNK)], sem))
        for c in copies:
            c.wait()

        orow_off = lax.shift_right_logical(iota, 2)
        ocol_base = lax.bitwise_and(iota, 3) * row
        def body(b, _):
            rows = b * L + iota
            colbase = sub_v[pl.ds(b * L, L)] * row
            orow = b * gper + orow_off
            for c in range(row):
                x = plsc.load_gather(g_v, [rows, colbase + c])
                plsc.store_scatter(o_v, [orow, ocol_base + c], x)
            return 0
        lax.fori_loop(0, b_per_w // L, body, 0)

        pltpu.sync_copy(o_v, out_hbm.at[pl.ds(wid * or_per_w, or_per_w)])

    return k(ids, table)


def kernel(ids, boxes):
    num_models, num_boxes, two, dim = boxes.shape
    batch = ids.shape[0]
    row = num_models * two * dim
    table = boxes.reshape(num_boxes // 4, 4 * row)
    out = _gather(ids, table, batch)
    return out.reshape(num_models, batch, two, dim)


# speedup vs baseline: 1.0662x; 1.0662x over previous
import functools

import jax
import jax.numpy as jnp
from jax import lax
from jax.experimental import pallas as pl
from jax.experimental.pallas import tpu as pltpu
from jax.experimental.pallas import tpu_sc as plsc

NC = 2
NS = 16
NW = NC * NS


@jax.jit
def _gather(ids, table_t):
    n_coords = table_t.shape[0]
    batch = ids.shape[0]
    mesh = plsc.VectorSubcoreMesh(
        core_axis_name="c", subcore_axis_name="s",
        num_cores=NC, num_subcores=NS)

    @functools.partial(
        pl.kernel,
        out_type=jax.ShapeDtypeStruct((n_coords, batch), jnp.float32),
        mesh=mesh,
        scratch_types=[
            pltpu.VMEM((batch,), jnp.int32),
            pltpu.VMEM((batch,), jnp.float32),
            pltpu.SemaphoreType.DMA,
        ],
        compiler_params=pltpu.CompilerParams(use_tc_tiling_on_sc=False),
    )
    def k(ids_hbm, table_hbm, out_hbm, idx_v, o_v, sem):
        w = lax.axis_index("s") * NC + lax.axis_index("c")
        pltpu.sync_copy(ids_hbm, idx_v)
        pltpu.async_copy(table_hbm.at[w].at[idx_v], o_v, sem).wait()
        pltpu.sync_copy(o_v, out_hbm.at[w])

    return k(ids, table_t)


def kernel(ids, boxes):
    num_models, num_boxes, two, dim = boxes.shape
    batch = ids.shape[0]
    table_t = jnp.transpose(boxes, (0, 2, 3, 1)).reshape(
        num_models * two * dim, num_boxes)
    out2d = _gather(ids, table_t)
    return out2d.reshape(num_models, two, dim, batch).transpose(0, 3, 1, 2)


# speedup vs baseline: 16.6046x; 15.5743x over previous
import functools

import jax
import jax.numpy as jnp
from jax import lax
from jax.experimental import pallas as pl
from jax.experimental.pallas import tpu as pltpu
from jax.experimental.pallas import tpu_sc as plsc

NC = 2
NS = 16
NW = NC * NS
CW = 1024


def _mesh():
    return plsc.VectorSubcoreMesh(
        core_axis_name="c", subcore_axis_name="s",
        num_cores=NC, num_subcores=NS)


@jax.jit
def _linearize(table_t, tail_p):
    n_coords, n_boxes = table_t.shape
    tcols = -(-n_boxes // 128)
    tc_per_w = -(-tcols // NW)
    slab = tc_per_w * 128
    n_chunks = -(-slab // CW)
    aligned = (n_boxes // 128) * 128
    stride = aligned + (128 if n_boxes > aligned else 0)

    @functools.partial(
        pl.kernel,
        out_type=jax.ShapeDtypeStruct((n_coords * stride,), jnp.float32),
        mesh=_mesh(),
        scratch_types=[
            pltpu.VMEM((n_coords, CW), jnp.float32),
            pltpu.VMEM((n_coords, 128), jnp.float32),
            pltpu.SemaphoreType.DMA,
        ],
    )
    def k(table_hbm, tail_hbm, out_hbm, buf_v, tbuf_v, sem):
        w = lax.axis_index("s") * NC + lax.axis_index("c")
        base = w * slab
        hi = aligned - CW
        def body(j, _):
            off = jnp.minimum(base + j * CW, hi)
            off = pl.multiple_of(off, 128)
            pltpu.sync_copy(table_hbm.at[:, pl.ds(off, CW)], buf_v)
            copies = [
                pltpu.async_copy(
                    buf_v.at[r], out_hbm.at[pl.ds(r * stride + off, CW)],
                    sem)
                for r in range(n_coords)
            ]
            for c in copies:
                c.wait()
            return 0
        lax.fori_loop(0, n_chunks, body, 0)
        if stride > aligned:
            @pl.when(w == NW - 1)
            def _():
                pltpu.sync_copy(tail_hbm, tbuf_v)
                copies = [
                    pltpu.async_copy(
                        tbuf_v.at[r],
                        out_hbm.at[pl.ds(r * stride + aligned, 128)], sem)
                    for r in range(n_coords)
                ]
                for c in copies:
                    c.wait()

    return k(table_t, tail_p)


@jax.jit
def _gather(ids, table_lin2d):
    n_coords = table_lin2d.shape[0]
    batch = ids.shape[0]

    @functools.partial(
        pl.kernel,
        out_type=jax.ShapeDtypeStruct((n_coords * batch,), jnp.float32),
        mesh=_mesh(),
        scratch_types=[
            pltpu.VMEM((batch,), jnp.int32),
            pltpu.VMEM((batch,), jnp.float32),
            pltpu.SemaphoreType.DMA,
        ],
        compiler_params=pltpu.CompilerParams(use_tc_tiling_on_sc=False),
    )
    def k(ids_hbm, table_hbm, out_hbm, idx_v, o_v, sem):
        w = lax.axis_index("s") * NC + lax.axis_index("c")
        pltpu.sync_copy(ids_hbm, idx_v)
        pltpu.async_copy(table_hbm.at[w].at[idx_v], o_v, sem).wait()
        pltpu.sync_copy(o_v, out_hbm.at[pl.ds(w * batch, batch)])

    return k(ids, table_lin2d)


def kernel(ids, boxes):
    num_models, num_boxes, two, dim = boxes.shape
    batch = ids.shape[0]
    n_coords = num_models * two * dim
    table_t = jnp.transpose(boxes, (0, 2, 3, 1)).reshape(n_coords, num_boxes)
    aligned = (num_boxes // 128) * 128
    stride = aligned + (128 if num_boxes > aligned else 0)
    tail_p = jnp.pad(table_t[:, aligned:], ((0, 0), (0, stride - num_boxes)))
    table_lin = _linearize(table_t, tail_p).reshape(n_coords, stride)
    out_flat = _gather(ids, table_lin)
    return out_flat.reshape(num_models, two, dim, batch).transpose(0, 3, 1, 2)
